# Initial kernel scaffold; baseline (speedup 1.0000x reference)
import jax
import jax.numpy as jnp
from jax.experimental import pallas as pl


def kernel(xyz, cls_label, params):
    raise NotImplementedError("write your pallas kernel here")


# speedup vs baseline: 1.1812x; 1.1812x over previous
import jax
import jax.numpy as jnp
import numpy as np
from jax.experimental import pallas as pl
from jax.experimental.pallas import tpu as pltpu

_BN_EPS = 1e-5


def _fold_layer(p):
    s = p["gamma"] / jnp.sqrt(1.0 + _BN_EPS)
    wt = (p["W"] * s[:, None]).T
    b = (p["b"] * s + p["beta"]).reshape(1, -1)
    return wt, b


def _mlp_rows(h, wrefs, nlayers):
    for i in range(nlayers):
        wt = wrefs[2 * i][...]
        b = wrefs[2 * i + 1][...]
        h = jnp.dot(h, wt, preferred_element_type=jnp.float32) + b
        h = jnp.maximum(h, 0.0)
    return h


def _sa_mlp_max(x, layers, s_tile):
    B, S, K, C = x.shape
    nlayers = len(layers)
    c_last = layers[-1]["W"].shape[0]
    wb = []
    for p in layers:
        wt, b = _fold_layer(p)
        wb += [wt, b]

    def kern(x_ref, *refs):
        out_ref = refs[-1]
        h = x_ref[0].reshape(s_tile * K, C)
        h = _mlp_rows(h, refs[:-1], nlayers)
        out_ref[0] = h.reshape(s_tile, K, c_last).max(axis=1)

    in_specs = [pl.BlockSpec((1, s_tile, K, C), lambda b, s: (b, s, 0, 0))]
    for a in wb:
        in_specs.append(pl.BlockSpec(a.shape, lambda b, s: (0,) * a.ndim))
    return pl.pallas_call(
        kern,
        grid=(B, S // s_tile),
        in_specs=in_specs,
        out_specs=pl.BlockSpec((1, s_tile, c_last), lambda b, s: (b, s, 0)),
        out_shape=jax.ShapeDtypeStruct((B, S, c_last), jnp.float32),
    )(x, *wb)


def _pt_mlp(x, layers, n_tile):
    B, N, C = x.shape
    nlayers = len(layers)
    c_last = layers[-1]["W"].shape[0]
    wb = []
    for p in layers:
        wt, b = _fold_layer(p)
        wb += [wt, b]

    def kern(x_ref, *refs):
        out_ref = refs[-1]
        out_ref[0] = _mlp_rows(x_ref[0], refs[:-1], nlayers)

    in_specs = [pl.BlockSpec((1, n_tile, C), lambda b, n: (b, n, 0))]
    for a in wb:
        in_specs.append(pl.BlockSpec(a.shape, lambda b, n: (0,) * a.ndim))
    return pl.pallas_call(
        kern,
        grid=(B, N // n_tile),
        in_specs=in_specs,
        out_specs=pl.BlockSpec((1, n_tile, c_last), lambda b, n: (b, n, 0)),
        out_shape=jax.ShapeDtypeStruct((B, N, c_last), jnp.float32),
    )(x, *wb)


def _head(x, conv1, conv2, n_tile):
    B, N, C = x.shape
    w1, b1 = _fold_layer(conv1)
    w2 = conv2["W"].T
    b2 = conv2["b"].reshape(1, -1)
    c_out = w2.shape[1]

    def kern(x_ref, w1_ref, b1_ref, w2_ref, b2_ref, out_ref):
        h = x_ref[0]
        h = jnp.maximum(
            jnp.dot(h, w1_ref[...], preferred_element_type=jnp.float32) + b1_ref[...],
            0.0,
        )
        logits = jnp.dot(h, w2_ref[...], preferred_element_type=jnp.float32) + b2_ref[...]
        m = jnp.max(logits, axis=-1, keepdims=True)
        sh = logits - m
        lse = jnp.log(jnp.sum(jnp.exp(sh), axis=-1, keepdims=True))
        out_ref[0] = sh - lse

    in_specs = [pl.BlockSpec((1, n_tile, C), lambda b, n: (b, n, 0))]
    for a in (w1, b1, w2, b2):
        in_specs.append(pl.BlockSpec(a.shape, lambda b, n: (0,) * a.ndim))
    return pl.pallas_call(
        kern,
        grid=(B, N // n_tile),
        in_specs=in_specs,
        out_specs=pl.BlockSpec((1, n_tile, c_out), lambda b, n: (b, n, 0)),
        out_shape=jax.ShapeDtypeStruct((B, N, c_out), jnp.float32),
    )(x, w1, b1, w2, b2)


def _sqdist(src, dst):
    d = -2.0 * jnp.einsum("bnc,bmc->bnm", src, dst)
    d = d + jnp.sum(src**2, -1)[:, :, None]
    d = d + jnp.sum(dst**2, -1)[:, None, :]
    return d


def _gather_pts(points, idx):
    return jax.vmap(lambda p, i: p[i])(points, idx)


def _fps(xyz, npoint):
    B, N, _ = xyz.shape

    def body(i, state):
        centroids, distance, farthest = state
        centroids = centroids.at[:, i].set(farthest)
        centroid = _gather_pts(xyz, farthest[:, None])
        dist = jnp.sum((xyz - centroid) ** 2, -1)
        distance = jnp.minimum(distance, dist)
        farthest = jnp.argmax(distance, -1).astype(jnp.int32)
        return (centroids, distance, farthest)

    centroids = jnp.zeros((B, npoint), jnp.int32)
    distance = jnp.full((B, N), 1e10, jnp.float32)
    farthest = jnp.zeros((B,), jnp.int32)
    centroids, _, _ = jax.lax.fori_loop(0, npoint, body, (centroids, distance, farthest))
    return centroids


def _ball_query(radius, nsample, xyz, new_xyz):
    B, N, _ = xyz.shape
    S = new_xyz.shape[1]
    sqrdists = _sqdist(new_xyz, xyz)
    group_idx = jnp.broadcast_to(jnp.arange(N, dtype=jnp.int32), (B, S, N))
    group_idx = jnp.where(sqrdists > radius**2, N, group_idx)
    group_idx = jnp.sort(group_idx, axis=-1)[:, :, :nsample]
    group_first = jnp.broadcast_to(group_idx[:, :, :1], group_idx.shape)
    group_idx = jnp.where(group_idx == N, group_first, group_idx)
    return group_idx


def _three_interp(xyz1, xyz2, points2):
    dists = _sqdist(xyz1, xyz2)
    neg_d, idx = jax.lax.top_k(-dists, 3)
    d = -neg_d
    recip = 1.0 / (d + 1e-8)
    weight = recip / jnp.sum(recip, axis=2, keepdims=True)
    return jnp.sum(_gather_pts(points2, idx) * weight[..., None], axis=2)


def _set_abstraction(xyz, points, layers, npoint, radius, nsample, s_tile):
    fps_idx = _fps(xyz, npoint)
    new_xyz = _gather_pts(xyz, fps_idx)
    idx = _ball_query(radius, nsample, xyz, new_xyz)
    grouped_xyz = _gather_pts(xyz, idx) - new_xyz[:, :, None, :]
    grouped_points = _gather_pts(points, idx)
    grouped = jnp.concatenate([grouped_xyz, grouped_points], axis=-1)
    new_points = _sa_mlp_max(grouped, layers, s_tile)
    return new_xyz, new_points


def kernel(xyz, cls_label, params):
    B, _, N = xyz.shape
    xyz_t = jnp.transpose(xyz, (0, 2, 1))
    l0_points = xyz_t

    l1_xyz, l1_points = _set_abstraction(
        xyz_t, l0_points, params["sa1"], 512, 0.2, 32, s_tile=128
    )
    l2_xyz, l2_points = _set_abstraction(
        l1_xyz, l1_points, params["sa2"], 128, 0.4, 64, s_tile=32
    )
    g3 = jnp.concatenate([l2_xyz, l2_points], axis=-1)[:, None, :, :]
    l3_points = _sa_mlp_max(g3, params["sa3"], s_tile=1)

    interp = jnp.broadcast_to(l3_points, (B, 128, 1024))
    f = jnp.concatenate([l2_points, interp], axis=-1)
    l2_points = _pt_mlp(f, params["fp3"], n_tile=128)

    interp = _three_interp(l1_xyz, l2_xyz, l2_points)
    f = jnp.concatenate([l1_points, interp], axis=-1)
    l1_points = _pt_mlp(f, params["fp2"], n_tile=512)

    interp = _three_interp(xyz_t, l1_xyz, l1_points)
    cls_oh = jnp.broadcast_to(cls_label[:, None, :], (B, N, 16))
    f = jnp.concatenate([cls_oh, xyz_t, l0_points, interp], axis=-1)
    l0 = _pt_mlp(f, params["fp1"], n_tile=512)

    x = _head(l0, params["conv1"], params["conv2"], n_tile=512)
    return x, jnp.transpose(l3_points, (0, 2, 1))


# speedup vs baseline: 1.5544x; 1.3159x over previous
import jax
import jax.numpy as jnp
import numpy as np
from jax.experimental import pallas as pl
from jax.experimental.pallas import tpu as pltpu

_BN_EPS = 1e-5


def _fold_layer(p):
    s = p["gamma"] / jnp.sqrt(1.0 + _BN_EPS)
    wt = (p["W"] * s[:, None]).T
    b = (p["b"] * s + p["beta"]).reshape(1, -1)
    return wt, b


def _mlp_rows(h, wrefs, nlayers):
    for i in range(nlayers):
        wt = wrefs[2 * i][...]
        b = wrefs[2 * i + 1][...]
        h = jnp.dot(h, wt, preferred_element_type=jnp.float32) + b
        h = jnp.maximum(h, 0.0)
    return h


def _sa_mlp_max(x, layers, s_tile):
    B, S, K, C = x.shape
    nlayers = len(layers)
    c_last = layers[-1]["W"].shape[0]
    wb = []
    for p in layers:
        wt, b = _fold_layer(p)
        wb += [wt, b]

    def kern(x_ref, *refs):
        out_ref = refs[-1]
        h = x_ref[0].reshape(s_tile * K, C)
        h = _mlp_rows(h, refs[:-1], nlayers)
        out_ref[0] = h.reshape(s_tile, K, c_last).max(axis=1)

    in_specs = [pl.BlockSpec((1, s_tile, K, C), lambda b, s: (b, s, 0, 0))]
    for a in wb:
        in_specs.append(pl.BlockSpec(a.shape, lambda b, s: (0,) * a.ndim))
    return pl.pallas_call(
        kern,
        grid=(B, S // s_tile),
        in_specs=in_specs,
        out_specs=pl.BlockSpec((1, s_tile, c_last), lambda b, s: (b, s, 0)),
        out_shape=jax.ShapeDtypeStruct((B, S, c_last), jnp.float32),
    )(x, *wb)


def _pt_mlp(x, layers, n_tile):
    B, N, C = x.shape
    nlayers = len(layers)
    c_last = layers[-1]["W"].shape[0]
    wb = []
    for p in layers:
        wt, b = _fold_layer(p)
        wb += [wt, b]

    def kern(x_ref, *refs):
        out_ref = refs[-1]
        out_ref[0] = _mlp_rows(x_ref[0], refs[:-1], nlayers)

    in_specs = [pl.BlockSpec((1, n_tile, C), lambda b, n: (b, n, 0))]
    for a in wb:
        in_specs.append(pl.BlockSpec(a.shape, lambda b, n: (0,) * a.ndim))
    return pl.pallas_call(
        kern,
        grid=(B, N // n_tile),
        in_specs=in_specs,
        out_specs=pl.BlockSpec((1, n_tile, c_last), lambda b, n: (b, n, 0)),
        out_shape=jax.ShapeDtypeStruct((B, N, c_last), jnp.float32),
    )(x, *wb)


def _head(x, conv1, conv2, n_tile):
    B, N, C = x.shape
    w1, b1 = _fold_layer(conv1)
    w2 = conv2["W"].T
    b2 = conv2["b"].reshape(1, -1)
    c_out = w2.shape[1]

    def kern(x_ref, w1_ref, b1_ref, w2_ref, b2_ref, out_ref):
        h = x_ref[0]
        h = jnp.maximum(
            jnp.dot(h, w1_ref[...], preferred_element_type=jnp.float32) + b1_ref[...],
            0.0,
        )
        logits = jnp.dot(h, w2_ref[...], preferred_element_type=jnp.float32) + b2_ref[...]
        m = jnp.max(logits, axis=-1, keepdims=True)
        sh = logits - m
        lse = jnp.log(jnp.sum(jnp.exp(sh), axis=-1, keepdims=True))
        out_ref[0] = sh - lse

    in_specs = [pl.BlockSpec((1, n_tile, C), lambda b, n: (b, n, 0))]
    for a in (w1, b1, w2, b2):
        in_specs.append(pl.BlockSpec(a.shape, lambda b, n: (0,) * a.ndim))
    return pl.pallas_call(
        kern,
        grid=(B, N // n_tile),
        in_specs=in_specs,
        out_specs=pl.BlockSpec((1, n_tile, c_out), lambda b, n: (b, n, 0)),
        out_shape=jax.ShapeDtypeStruct((B, N, c_out), jnp.float32),
    )(x, w1, b1, w2, b2)


def _sqdist(src, dst):
    d = -2.0 * jnp.einsum("bnc,bmc->bnm", src, dst)
    d = d + jnp.sum(src**2, -1)[:, :, None]
    d = d + jnp.sum(dst**2, -1)[:, None, :]
    return d


def _gather_pts(points, idx):
    return jax.vmap(lambda p, i: p[i])(points, idx)


def _fps(xyz, npoint):
    B, N, _ = xyz.shape
    planes = jnp.transpose(xyz, (2, 0, 1))

    def kern(x_ref, y_ref, z_ref, ox_ref, oy_ref, oz_ref, dist_ref):
        x = x_ref[...]
        y = y_ref[...]
        z = z_ref[...]
        iota = jax.lax.broadcasted_iota(jnp.int32, (B, N), 1)
        iota_s = jax.lax.broadcasted_iota(jnp.int32, (B, npoint), 1)
        dist_ref[...] = jnp.full((B, N), 1e10, jnp.float32)

        def body(i, carry):
            cx, cy, cz = carry
            col = (iota_s == i).astype(jnp.float32)
            ox_ref[...] += col * cx
            oy_ref[...] += col * cy
            oz_ref[...] += col * cz
            d = (x - cx) ** 2 + (y - cy) ** 2 + (z - cz) ** 2
            dist = jnp.minimum(dist_ref[...], d)
            dist_ref[...] = dist
            m = jnp.max(dist, axis=1, keepdims=True)
            sel = jnp.where(dist == m, iota, jnp.int32(N))
            idx = jnp.min(sel, axis=1, keepdims=True)
            oh = (iota == idx).astype(jnp.float32)
            ncx = jnp.sum(oh * x, axis=1, keepdims=True)
            ncy = jnp.sum(oh * y, axis=1, keepdims=True)
            ncz = jnp.sum(oh * z, axis=1, keepdims=True)
            return (ncx, ncy, ncz)

        ox_ref[...] = jnp.zeros((B, npoint), jnp.float32)
        oy_ref[...] = jnp.zeros((B, npoint), jnp.float32)
        oz_ref[...] = jnp.zeros((B, npoint), jnp.float32)
        jax.lax.fori_loop(
            0, npoint, body, (x[:, 0:1], y[:, 0:1], z[:, 0:1]), unroll=False
        )

    ox, oy, oz = pl.pallas_call(
        kern,
        out_shape=[jax.ShapeDtypeStruct((B, npoint), jnp.float32)] * 3,
        scratch_shapes=[pltpu.VMEM((B, N), jnp.float32)],
    )(planes[0], planes[1], planes[2])
    return jnp.stack([ox, oy, oz], axis=-1)


def _ball_query(radius, nsample, xyz, new_xyz):
    B, N, _ = xyz.shape
    S = new_xyz.shape[1]
    sqrdists = _sqdist(new_xyz, xyz)
    group_idx = jnp.broadcast_to(jnp.arange(N, dtype=jnp.int32), (B, S, N))
    group_idx = jnp.where(sqrdists > radius**2, N, group_idx)
    group_idx = jnp.sort(group_idx, axis=-1)[:, :, :nsample]
    group_first = jnp.broadcast_to(group_idx[:, :, :1], group_idx.shape)
    group_idx = jnp.where(group_idx == N, group_first, group_idx)
    return group_idx


def _three_interp(xyz1, xyz2, points2):
    dists = _sqdist(xyz1, xyz2)
    neg_d, idx = jax.lax.top_k(-dists, 3)
    d = -neg_d
    recip = 1.0 / (d + 1e-8)
    weight = recip / jnp.sum(recip, axis=2, keepdims=True)
    return jnp.sum(_gather_pts(points2, idx) * weight[..., None], axis=2)


def _set_abstraction(xyz, points, layers, npoint, radius, nsample, s_tile):
    new_xyz = _fps(xyz, npoint)
    idx = _ball_query(radius, nsample, xyz, new_xyz)
    grouped_xyz = _gather_pts(xyz, idx) - new_xyz[:, :, None, :]
    grouped_points = _gather_pts(points, idx)
    grouped = jnp.concatenate([grouped_xyz, grouped_points], axis=-1)
    new_points = _sa_mlp_max(grouped, layers, s_tile)
    return new_xyz, new_points


def kernel(xyz, cls_label, params):
    B, _, N = xyz.shape
    xyz_t = jnp.transpose(xyz, (0, 2, 1))
    l0_points = xyz_t

    l1_xyz, l1_points = _set_abstraction(
        xyz_t, l0_points, params["sa1"], 512, 0.2, 32, s_tile=128
    )
    l2_xyz, l2_points = _set_abstraction(
        l1_xyz, l1_points, params["sa2"], 128, 0.4, 64, s_tile=32
    )
    g3 = jnp.concatenate([l2_xyz, l2_points], axis=-1)[:, None, :, :]
    l3_points = _sa_mlp_max(g3, params["sa3"], s_tile=1)

    interp = jnp.broadcast_to(l3_points, (B, 128, 1024))
    f = jnp.concatenate([l2_points, interp], axis=-1)
    l2_points = _pt_mlp(f, params["fp3"], n_tile=128)

    interp = _three_interp(l1_xyz, l2_xyz, l2_points)
    f = jnp.concatenate([l1_points, interp], axis=-1)
    l1_points = _pt_mlp(f, params["fp2"], n_tile=512)

    interp = _three_interp(xyz_t, l1_xyz, l1_points)
    cls_oh = jnp.broadcast_to(cls_label[:, None, :], (B, N, 16))
    f = jnp.concatenate([cls_oh, xyz_t, l0_points, interp], axis=-1)
    l0 = _pt_mlp(f, params["fp1"], n_tile=512)

    x = _head(l0, params["conv1"], params["conv2"], n_tile=512)
    return x, jnp.transpose(l3_points, (0, 2, 1))


# speedup vs baseline: 8.4355x; 5.4267x over previous
import jax
import jax.numpy as jnp
import numpy as np
from jax.experimental import pallas as pl
from jax.experimental.pallas import tpu as pltpu

_BN_EPS = 1e-5


def _fold_layer(p):
    s = p["gamma"] / jnp.sqrt(1.0 + _BN_EPS)
    wt = (p["W"] * s[:, None]).T
    b = (p["b"] * s + p["beta"]).reshape(1, -1)
    return wt, b


def _mlp_rows(h, wrefs, nlayers):
    for i in range(nlayers):
        wt = wrefs[2 * i][...]
        b = wrefs[2 * i + 1][...]
        h = jnp.dot(h, wt, preferred_element_type=jnp.float32) + b
        h = jnp.maximum(h, 0.0)
    return h


def _sqdist_rows(a, bt):
    e = jnp.dot(a, bt, preferred_element_type=jnp.float32)
    n1 = a[:, 0:1] ** 2 + a[:, 1:2] ** 2 + a[:, 2:3] ** 2
    n2 = bt[0:1, :] ** 2 + bt[1:2, :] ** 2 + bt[2:3, :] ** 2
    return -2.0 * e + n1 + n2


def _topk3_weighted_onehot(d, n_rows, S):
    iota = jax.lax.broadcasted_iota(jnp.int32, (n_rows, S), 1)
    dcur = d
    oh = jnp.zeros((n_rows, S), jnp.float32)
    rsum = jnp.zeros((n_rows, 1), jnp.float32)
    for _ in range(3):
        mj = jnp.min(dcur, axis=1, keepdims=True)
        ij = jnp.min(jnp.where(dcur == mj, iota, jnp.int32(S)), axis=1, keepdims=True)
        ohj = iota == ij
        recipj = 1.0 / (mj + 1e-8)
        oh = oh + recipj * ohj.astype(jnp.float32)
        rsum = rsum + recipj
        dcur = jnp.where(ohj, jnp.float32(np.inf), dcur)
    return oh / rsum


def _fps(xyz, npoint):
    B, N, _ = xyz.shape
    planes = jnp.transpose(xyz, (2, 0, 1))

    def kern(x_ref, y_ref, z_ref, ox_ref, oy_ref, oz_ref, dist_ref):
        x = x_ref[...]
        y = y_ref[...]
        z = z_ref[...]
        iota = jax.lax.broadcasted_iota(jnp.int32, (B, N), 1)
        iota_s = jax.lax.broadcasted_iota(jnp.int32, (B, npoint), 1)
        dist_ref[...] = jnp.full((B, N), 1e10, jnp.float32)

        def body(i, carry):
            cx, cy, cz = carry
            col = (iota_s == i).astype(jnp.float32)
            ox_ref[...] += col * cx
            oy_ref[...] += col * cy
            oz_ref[...] += col * cz
            d = (x - cx) ** 2 + (y - cy) ** 2 + (z - cz) ** 2
            dist = jnp.minimum(dist_ref[...], d)
            dist_ref[...] = dist
            m = jnp.max(dist, axis=1, keepdims=True)
            sel = jnp.where(dist == m, iota, jnp.int32(N))
            idx = jnp.min(sel, axis=1, keepdims=True)
            oh = (iota == idx).astype(jnp.float32)
            ncx = jnp.sum(oh * x, axis=1, keepdims=True)
            ncy = jnp.sum(oh * y, axis=1, keepdims=True)
            ncz = jnp.sum(oh * z, axis=1, keepdims=True)
            return (ncx, ncy, ncz)

        ox_ref[...] = jnp.zeros((B, npoint), jnp.float32)
        oy_ref[...] = jnp.zeros((B, npoint), jnp.float32)
        oz_ref[...] = jnp.zeros((B, npoint), jnp.float32)
        jax.lax.fori_loop(
            0, npoint, body, (x[:, 0:1], y[:, 0:1], z[:, 0:1]), unroll=False
        )

    ox, oy, oz = pl.pallas_call(
        kern,
        out_shape=[jax.ShapeDtypeStruct((B, npoint), jnp.float32)] * 3,
        scratch_shapes=[pltpu.VMEM((B, N), jnp.float32)],
    )(planes[0], planes[1], planes[2])
    return jnp.stack([ox, oy, oz], axis=-1)


def _sa_group_mlp_max(pts, pts3t, new_xyz, sub, layers, radius, nsample, s_tile):
    B, N, C = pts.shape
    S = new_xyz.shape[1]
    K = nsample
    nlayers = len(layers)
    c_last = layers[-1]["W"].shape[0]
    r2 = float(radius) * float(radius)
    nchunk = N // 128
    tri = jnp.asarray(np.triu(np.ones((128, 128), np.float32)))
    tri_c = jnp.asarray(np.triu(np.ones((nchunk, nchunk), np.float32), k=1))
    wb = []
    for p in layers:
        wt, b = _fold_layer(p)
        wb += [wt, b]

    def kern(pts_ref, p3t_ref, nx_ref, sub_ref, tri_ref, tric_ref, *rest):
        wrefs = rest[:-3]
        out_ref = rest[-3]
        m_ref = rest[-2]
        rank_ref = rest[-1]
        cen = nx_ref[0]
        p3t = p3t_ref[0]
        d = _sqdist_rows(cen, p3t)
        u = (d <= r2).astype(jnp.float32)
        tri_m = tri_ref[...]
        tots = []
        for c in range(nchunk):
            ic = jnp.dot(
                u[:, c * 128 : (c + 1) * 128], tri_m,
                preferred_element_type=jnp.float32,
            )
            rank_ref[:, c * 128 : (c + 1) * 128] = ic
            tots.append(ic[:, 127:128])
        totals = jnp.concatenate(tots, axis=1)
        carry = jnp.dot(totals, tric_ref[...], preferred_element_type=jnp.float32)
        for c in range(nchunk):
            rank_ref[:, c * 128 : (c + 1) * 128] += carry[:, c : c + 1]
        cnt = carry[:, nchunk - 1 : nchunk] + totals[:, nchunk - 1 : nchunk]
        rank = rank_ref[...]
        first = u * (rank == 1.0).astype(jnp.float32)
        lastcol = (
            jax.lax.broadcasted_iota(jnp.int32, (s_tile, N), 1) == N - 1
        ).astype(jnp.float32)
        first = jnp.where(cnt >= 1.0, first, lastcol)
        for k in range(K):
            selk = u * (rank == jnp.float32(k + 1)).astype(jnp.float32)
            selk = jnp.where(cnt >= jnp.float32(k + 1), selk, first)
            m_ref[k * s_tile : (k + 1) * s_tile, :] = selk
        grouped = jnp.dot(
            m_ref[...],
            pts_ref[0],
            preferred_element_type=jnp.float32,
            precision=jax.lax.Precision.HIGHEST,
        )
        g3 = grouped.reshape(K, s_tile, C) - sub_ref[0][None, :, :]
        h = _mlp_rows(g3.reshape(K * s_tile, C), wrefs, nlayers)
        out_ref[0] = jnp.max(h.reshape(K, s_tile, c_last), axis=0)

    in_specs = [
        pl.BlockSpec((1, N, C), lambda b, s: (b, 0, 0)),
        pl.BlockSpec((1, 3, N), lambda b, s: (b, 0, 0)),
        pl.BlockSpec((1, s_tile, 3), lambda b, s: (b, s, 0)),
        pl.BlockSpec((1, s_tile, C), lambda b, s: (b, s, 0)),
        pl.BlockSpec((128, 128), lambda b, s: (0, 0)),
        pl.BlockSpec((nchunk, nchunk), lambda b, s: (0, 0)),
    ]
    for a in wb:
        in_specs.append(pl.BlockSpec(a.shape, lambda b, s: (0,) * a.ndim))
    return pl.pallas_call(
        kern,
        grid=(B, S // s_tile),
        in_specs=in_specs,
        out_specs=pl.BlockSpec((1, s_tile, c_last), lambda b, s: (b, s, 0)),
        out_shape=jax.ShapeDtypeStruct((B, S, c_last), jnp.float32),
        scratch_shapes=[
            pltpu.VMEM((K * s_tile, N), jnp.float32),
            pltpu.VMEM((s_tile, N), jnp.float32),
        ],
    )(pts, pts3t, new_xyz, sub, tri, tri_c, *wb)


def _sa_all_mlp_max(x, layers):
    B, K, C = x.shape
    nlayers = len(layers)
    c_last = layers[-1]["W"].shape[0]
    wb = []
    for p in layers:
        wt, b = _fold_layer(p)
        wb += [wt, b]

    def kern(x_ref, *refs):
        out_ref = refs[-1]
        h = _mlp_rows(x_ref[0], refs[:-1], nlayers)
        out_ref[0] = jnp.max(h, axis=0, keepdims=True)

    in_specs = [pl.BlockSpec((1, K, C), lambda b: (b, 0, 0))]
    for a in wb:
        in_specs.append(pl.BlockSpec(a.shape, lambda b: (0,) * a.ndim))
    return pl.pallas_call(
        kern,
        grid=(B,),
        in_specs=in_specs,
        out_specs=pl.BlockSpec((1, 1, c_last), lambda b: (b, 0, 0)),
        out_shape=jax.ShapeDtypeStruct((B, 1, c_last), jnp.float32),
    )(x, *wb)


def _fp_interp_mlp(xyz1, xyz2t, points2, points1, layers, n_tile, extra=None):
    B, N, _ = xyz1.shape
    S = xyz2t.shape[2]
    C1 = points1.shape[2]
    C2 = points2.shape[2]
    nlayers = len(layers)
    c_last = layers[-1]["W"].shape[0]
    if isinstance(layers[0], tuple):
        w1a, w1b, b1 = layers[0]
    else:
        w0, b1 = _fold_layer(layers[0])
        w1a, w1b = w0[:C1], w0[C1:]
    c0 = w1a.shape[1]
    wb = []
    for p in layers[1:]:
        wt, b = _fold_layer(p)
        wb += [wt, b]
    has_extra = extra is not None

    def kern(*args):
        base = 7 + (1 if has_extra else 0)
        x1_ref, x2t_ref, p2_ref, p1_ref, w1a_ref, w1b_ref, b1_ref = args[:7]
        refs = args[base:]
        out_ref = refs[-1]
        wrefs = refs[:-1]
        x1 = x1_ref[0]
        x2t = x2t_ref[0]
        d = _sqdist_rows(x1, x2t)
        oh = _topk3_weighted_onehot(d, n_tile, S)
        interp = jnp.dot(
            oh,
            p2_ref[0],
            preferred_element_type=jnp.float32,
            precision=jax.lax.Precision.HIGHEST,
        )
        h = (
            jnp.dot(p1_ref[0], w1a_ref[...], preferred_element_type=jnp.float32)
            + jnp.dot(interp, w1b_ref[...], preferred_element_type=jnp.float32)
            + b1_ref[...]
        )
        if has_extra:
            h = h + args[7][0]
        h = jnp.maximum(h, 0.0)
        out_ref[0] = _mlp_rows(h, wrefs, nlayers - 1)

    in_specs = [
        pl.BlockSpec((1, n_tile, 3), lambda b, n: (b, n, 0)),
        pl.BlockSpec((1, 3, S), lambda b, n: (b, 0, 0)),
        pl.BlockSpec((1, S, C2), lambda b, n: (b, 0, 0)),
        pl.BlockSpec((1, n_tile, C1), lambda b, n: (b, n, 0)),
        pl.BlockSpec(w1a.shape, lambda b, n: (0, 0)),
        pl.BlockSpec(w1b.shape, lambda b, n: (0, 0)),
        pl.BlockSpec(b1.shape, lambda b, n: (0, 0)),
    ]
    operands = [xyz1, xyz2t, points2, points1, w1a, w1b, b1]
    if has_extra:
        in_specs.append(pl.BlockSpec((1, 1, c0), lambda b, n: (b, 0, 0)))
        operands.append(extra)
    for a in wb:
        in_specs.append(pl.BlockSpec(a.shape, lambda b, n: (0,) * a.ndim))
    operands += wb
    return pl.pallas_call(
        kern,
        grid=(B, N // n_tile),
        in_specs=in_specs,
        out_specs=pl.BlockSpec((1, n_tile, c_last), lambda b, n: (b, n, 0)),
        out_shape=jax.ShapeDtypeStruct((B, N, c_last), jnp.float32),
    )(*operands)


def _fp_broadcast_mlp(points2, points1, layers, n_tile):
    B, N, C1 = points1.shape
    C2 = points2.shape[2]
    nlayers = len(layers)
    c_last = layers[-1]["W"].shape[0]
    w0, b1 = _fold_layer(layers[0])
    w1a, w1b = w0[:C1], w0[C1:]
    wb = []
    for p in layers[1:]:
        wt, b = _fold_layer(p)
        wb += [wt, b]

    def kern(p2_ref, p1_ref, w1a_ref, w1b_ref, b1_ref, *refs):
        out_ref = refs[-1]
        wrefs = refs[:-1]
        t = jnp.dot(p2_ref[0], w1b_ref[...], preferred_element_type=jnp.float32)
        h = (
            jnp.dot(p1_ref[0], w1a_ref[...], preferred_element_type=jnp.float32)
            + t
            + b1_ref[...]
        )
        h = jnp.maximum(h, 0.0)
        out_ref[0] = _mlp_rows(h, wrefs, nlayers - 1)

    in_specs = [
        pl.BlockSpec((1, 1, C2), lambda b, n: (b, 0, 0)),
        pl.BlockSpec((1, n_tile, C1), lambda b, n: (b, n, 0)),
        pl.BlockSpec(w1a.shape, lambda b, n: (0, 0)),
        pl.BlockSpec(w1b.shape, lambda b, n: (0, 0)),
        pl.BlockSpec(b1.shape, lambda b, n: (0, 0)),
    ]
    for a in wb:
        in_specs.append(pl.BlockSpec(a.shape, lambda b, n: (0,) * a.ndim))
    return pl.pallas_call(
        kern,
        grid=(B, N // n_tile),
        in_specs=in_specs,
        out_specs=pl.BlockSpec((1, n_tile, c_last), lambda b, n: (b, n, 0)),
        out_shape=jax.ShapeDtypeStruct((B, N, c_last), jnp.float32),
    )(points2, points1, w1a, w1b, b1, *wb)


def _head(x, conv1, conv2, n_tile):
    B, N, C = x.shape
    w1, b1 = _fold_layer(conv1)
    w2 = conv2["W"].T
    b2 = conv2["b"].reshape(1, -1)
    c_out = w2.shape[1]

    def kern(x_ref, w1_ref, b1_ref, w2_ref, b2_ref, out_ref):
        h = x_ref[0]
        h = jnp.maximum(
            jnp.dot(h, w1_ref[...], preferred_element_type=jnp.float32) + b1_ref[...],
            0.0,
        )
        logits = jnp.dot(h, w2_ref[...], preferred_element_type=jnp.float32) + b2_ref[...]
        m = jnp.max(logits, axis=-1, keepdims=True)
        sh = logits - m
        lse = jnp.log(jnp.sum(jnp.exp(sh), axis=-1, keepdims=True))
        out_ref[0] = sh - lse

    in_specs = [pl.BlockSpec((1, n_tile, C), lambda b, n: (b, n, 0))]
    for a in (w1, b1, w2, b2):
        in_specs.append(pl.BlockSpec(a.shape, lambda b, n: (0,) * a.ndim))
    return pl.pallas_call(
        kern,
        grid=(B, N // n_tile),
        in_specs=in_specs,
        out_specs=pl.BlockSpec((1, n_tile, c_out), lambda b, n: (b, n, 0)),
        out_shape=jax.ShapeDtypeStruct((B, N, c_out), jnp.float32),
    )(x, w1, b1, w2, b2)


def kernel(xyz, cls_label, params):
    B, _, N = xyz.shape
    xyz_t = jnp.transpose(xyz, (0, 2, 1))

    new_xyz1 = _fps(xyz_t, 512)
    pts1 = jnp.concatenate([xyz_t, xyz_t], axis=-1)
    sub1 = jnp.concatenate([new_xyz1, jnp.zeros_like(new_xyz1)], axis=-1)
    l1_points = _sa_group_mlp_max(
        pts1, xyz, new_xyz1, sub1, params["sa1"], 0.2, 32, s_tile=16
    )

    new_xyz2 = _fps(new_xyz1, 128)
    pts2 = jnp.concatenate([new_xyz1, l1_points], axis=-1)
    sub2 = jnp.concatenate([new_xyz2, jnp.zeros((B, 128, 128), jnp.float32)], axis=-1)
    l2_points = _sa_group_mlp_max(
        pts2,
        jnp.transpose(new_xyz1, (0, 2, 1)),
        new_xyz2,
        sub2,
        params["sa2"],
        0.4,
        64,
        s_tile=16,
    )

    g3 = jnp.concatenate([new_xyz2, l2_points], axis=-1)
    l3_points = _sa_all_mlp_max(g3, params["sa3"])

    l2_points = _fp_broadcast_mlp(l3_points, l2_points, params["fp3"], n_tile=128)

    l1_points = _fp_interp_mlp(
        new_xyz1,
        jnp.transpose(new_xyz2, (0, 2, 1)),
        l2_points,
        l1_points,
        params["fp2"],
        n_tile=512,
    )

    w0, b0 = _fold_layer(params["fp1"][0])
    t_cls = jnp.einsum("bc,co->bo", cls_label, w0[:16])[:, None, :]
    prefolded = (w0[16:19] + w0[19:22], w0[22:], b0)
    l0 = _fp_interp_mlp(
        xyz_t,
        jnp.transpose(new_xyz1, (0, 2, 1)),
        l1_points,
        xyz_t,
        [prefolded] + params["fp1"][1:],
        n_tile=512,
        extra=t_cls,
    )

    x = _head(l0, params["conv1"], params["conv2"], n_tile=512)
    return x, jnp.transpose(l3_points, (0, 2, 1))


# speedup vs baseline: 9.2467x; 1.0962x over previous
import jax
import jax.numpy as jnp
import numpy as np
from jax.experimental import pallas as pl
from jax.experimental.pallas import tpu as pltpu

_BN_EPS = 1e-5


def _fold_layer(p):
    s = p["gamma"] / jnp.sqrt(1.0 + _BN_EPS)
    wt = (p["W"] * s[:, None]).T
    b = (p["b"] * s + p["beta"]).reshape(1, -1)
    return wt, b


def _mlp_rows(h, wrefs, nlayers):
    for i in range(nlayers):
        wt = wrefs[2 * i][...]
        b = wrefs[2 * i + 1][...]
        h = jnp.dot(h, wt, preferred_element_type=jnp.float32) + b
        h = jnp.maximum(h, 0.0)
    return h


def _sqdist_rows(a, bt):
    e = jnp.dot(a, bt, preferred_element_type=jnp.float32)
    n1 = a[:, 0:1] ** 2 + a[:, 1:2] ** 2 + a[:, 2:3] ** 2
    n2 = bt[0:1, :] ** 2 + bt[1:2, :] ** 2 + bt[2:3, :] ** 2
    return -2.0 * e + n1 + n2


def _topk3_weighted_onehot(d, n_rows, S):
    iota = jax.lax.broadcasted_iota(jnp.int32, (n_rows, S), 1)
    dcur = d
    oh = jnp.zeros((n_rows, S), jnp.float32)
    rsum = jnp.zeros((n_rows, 1), jnp.float32)
    for _ in range(3):
        mj = jnp.min(dcur, axis=1, keepdims=True)
        ij = jnp.min(jnp.where(dcur == mj, iota, jnp.int32(S)), axis=1, keepdims=True)
        ohj = iota == ij
        recipj = 1.0 / (mj + 1e-8)
        oh = oh + recipj * ohj.astype(jnp.float32)
        rsum = rsum + recipj
        dcur = jnp.where(ohj, jnp.float32(np.inf), dcur)
    return oh / rsum


def _fps(xyz, npoint):
    B, N, _ = xyz.shape
    planes = jnp.transpose(xyz, (2, 0, 1))

    def kern(x_ref, y_ref, z_ref, ox_ref, oy_ref, oz_ref, dist_ref):
        x = x_ref[...]
        y = y_ref[...]
        z = z_ref[...]
        iota = jax.lax.broadcasted_iota(jnp.int32, (B, N), 1)
        iota_s = jax.lax.broadcasted_iota(jnp.int32, (B, npoint), 1)
        dist_ref[...] = jnp.full((B, N), 1e10, jnp.float32)

        def body(i, carry):
            cx, cy, cz = carry
            col = (iota_s == i).astype(jnp.float32)
            ox_ref[...] += col * cx
            oy_ref[...] += col * cy
            oz_ref[...] += col * cz
            d = (x - cx) ** 2 + (y - cy) ** 2 + (z - cz) ** 2
            dist = jnp.minimum(dist_ref[...], d)
            dist_ref[...] = dist
            m = jnp.max(dist, axis=1, keepdims=True)
            sel = jnp.where(dist == m, iota, jnp.int32(N))
            idx = jnp.min(sel, axis=1, keepdims=True)
            oh = (iota == idx).astype(jnp.float32)
            ncx = jnp.sum(oh * x, axis=1, keepdims=True)
            ncy = jnp.sum(oh * y, axis=1, keepdims=True)
            ncz = jnp.sum(oh * z, axis=1, keepdims=True)
            return (ncx, ncy, ncz)

        ox_ref[...] = jnp.zeros((B, npoint), jnp.float32)
        oy_ref[...] = jnp.zeros((B, npoint), jnp.float32)
        oz_ref[...] = jnp.zeros((B, npoint), jnp.float32)
        jax.lax.fori_loop(
            0, npoint, body, (x[:, 0:1], y[:, 0:1], z[:, 0:1]), unroll=False
        )

    ox, oy, oz = pl.pallas_call(
        kern,
        out_shape=[jax.ShapeDtypeStruct((B, npoint), jnp.float32)] * 3,
        scratch_shapes=[pltpu.VMEM((B, N), jnp.float32)],
    )(planes[0], planes[1], planes[2])
    return jnp.stack([ox, oy, oz], axis=-1)


def _sa_group_mlp_max(pts, pts3t, new_xyz, sub, layers, radius, nsample, s_tile):
    B, N, C = pts.shape
    S = new_xyz.shape[1]
    K = nsample
    nlayers = len(layers)
    c_last = layers[-1]["W"].shape[0]
    r2 = float(radius) * float(radius)
    nchunk = N // 128
    tri = jnp.asarray(np.triu(np.ones((128, 128), np.float32)))
    tri_c = jnp.asarray(np.triu(np.ones((nchunk, nchunk), np.float32), k=1))
    wb = []
    for p in layers:
        wt, b = _fold_layer(p)
        wb += [wt, b]

    def kern(pts_ref, p3t_ref, nx_ref, sub_ref, tri_ref, tric_ref, *rest):
        wrefs = rest[:-3]
        out_ref = rest[-3]
        m_ref = rest[-2]
        rank_ref = rest[-1]
        cen = nx_ref[0]
        p3t = p3t_ref[0]
        d = _sqdist_rows(cen, p3t)
        u = (d <= r2).astype(jnp.float32)
        tri_m = tri_ref[...]
        tots = []
        for c in range(nchunk):
            ic = jnp.dot(
                u[:, c * 128 : (c + 1) * 128], tri_m,
                preferred_element_type=jnp.float32,
            )
            rank_ref[:, c * 128 : (c + 1) * 128] = ic
            tots.append(ic[:, 127:128])
        totals = jnp.concatenate(tots, axis=1)
        carry = jnp.dot(totals, tric_ref[...], preferred_element_type=jnp.float32)
        for c in range(nchunk):
            rank_ref[:, c * 128 : (c + 1) * 128] += carry[:, c : c + 1]
        cnt = carry[:, nchunk - 1 : nchunk] + totals[:, nchunk - 1 : nchunk]
        rank = rank_ref[...]
        first = u * (rank == 1.0).astype(jnp.float32)
        lastcol = (
            jax.lax.broadcasted_iota(jnp.int32, (s_tile, N), 1) == N - 1
        ).astype(jnp.float32)
        first = jnp.where(cnt >= 1.0, first, lastcol)
        for k in range(K):
            selk = u * (rank == jnp.float32(k + 1)).astype(jnp.float32)
            selk = jnp.where(cnt >= jnp.float32(k + 1), selk, first)
            m_ref[k * s_tile : (k + 1) * s_tile, :] = selk
        grouped = jnp.dot(
            m_ref[...],
            pts_ref[0],
            preferred_element_type=jnp.float32,
            precision=jax.lax.Precision.HIGHEST,
        )
        g3 = grouped.reshape(K, s_tile, C) - sub_ref[0][None, :, :]
        h = _mlp_rows(g3.reshape(K * s_tile, C), wrefs, nlayers)
        out_ref[0] = jnp.max(h.reshape(K, s_tile, c_last), axis=0)

    in_specs = [
        pl.BlockSpec((1, N, C), lambda b, s: (b, 0, 0)),
        pl.BlockSpec((1, 3, N), lambda b, s: (b, 0, 0)),
        pl.BlockSpec((1, s_tile, 3), lambda b, s: (b, s, 0)),
        pl.BlockSpec((1, s_tile, C), lambda b, s: (b, s, 0)),
        pl.BlockSpec((128, 128), lambda b, s: (0, 0)),
        pl.BlockSpec((nchunk, nchunk), lambda b, s: (0, 0)),
    ]
    for a in wb:
        in_specs.append(pl.BlockSpec(a.shape, lambda b, s: (0,) * a.ndim))
    return pl.pallas_call(
        kern,
        grid=(B, S // s_tile),
        in_specs=in_specs,
        out_specs=pl.BlockSpec((1, s_tile, c_last), lambda b, s: (b, s, 0)),
        out_shape=jax.ShapeDtypeStruct((B, S, c_last), jnp.float32),
        scratch_shapes=[
            pltpu.VMEM((K * s_tile, N), jnp.float32),
            pltpu.VMEM((s_tile, N), jnp.float32),
        ],
    )(pts, pts3t, new_xyz, sub, tri, tri_c, *wb)


def _sa_all_mlp_max(x, layers):
    B, K, C = x.shape
    nlayers = len(layers)
    c_last = layers[-1]["W"].shape[0]
    wb = []
    for p in layers:
        wt, b = _fold_layer(p)
        wb += [wt, b]

    def kern(x_ref, *refs):
        out_ref = refs[-1]
        h = _mlp_rows(x_ref[0], refs[:-1], nlayers)
        out_ref[0] = jnp.max(h, axis=0, keepdims=True)

    in_specs = [pl.BlockSpec((1, K, C), lambda b: (b, 0, 0))]
    for a in wb:
        in_specs.append(pl.BlockSpec(a.shape, lambda b: (0,) * a.ndim))
    return pl.pallas_call(
        kern,
        grid=(B,),
        in_specs=in_specs,
        out_specs=pl.BlockSpec((1, 1, c_last), lambda b: (b, 0, 0)),
        out_shape=jax.ShapeDtypeStruct((B, 1, c_last), jnp.float32),
    )(x, *wb)


def _fp_interp_mlp(xyz1, xyz2t, points2, points1, layers, n_tile, extra=None):
    B, N, _ = xyz1.shape
    S = xyz2t.shape[2]
    C1 = points1.shape[2]
    C2 = points2.shape[2]
    nlayers = len(layers)
    c_last = layers[-1]["W"].shape[0]
    if isinstance(layers[0], tuple):
        w1a, w1b, b1 = layers[0]
    else:
        w0, b1 = _fold_layer(layers[0])
        w1a, w1b = w0[:C1], w0[C1:]
    c0 = w1a.shape[1]
    wb = []
    for p in layers[1:]:
        wt, b = _fold_layer(p)
        wb += [wt, b]
    has_extra = extra is not None

    def kern(*args):
        base = 7 + (1 if has_extra else 0)
        x1_ref, x2t_ref, p2_ref, p1_ref, w1a_ref, w1b_ref, b1_ref = args[:7]
        refs = args[base:]
        out_ref = refs[-1]
        wrefs = refs[:-1]
        x1 = x1_ref[0]
        x2t = x2t_ref[0]
        d = _sqdist_rows(x1, x2t)
        oh = _topk3_weighted_onehot(d, n_tile, S)
        interp = jnp.dot(
            oh,
            p2_ref[0],
            preferred_element_type=jnp.float32,
            precision=jax.lax.Precision.HIGHEST,
        )
        h = (
            jnp.dot(p1_ref[0], w1a_ref[...], preferred_element_type=jnp.float32)
            + jnp.dot(interp, w1b_ref[...], preferred_element_type=jnp.float32)
            + b1_ref[...]
        )
        if has_extra:
            h = h + args[7][0]
        h = jnp.maximum(h, 0.0)
        out_ref[0] = _mlp_rows(h, wrefs, nlayers - 1)

    in_specs = [
        pl.BlockSpec((1, n_tile, 3), lambda b, n: (b, n, 0)),
        pl.BlockSpec((1, 3, S), lambda b, n: (b, 0, 0)),
        pl.BlockSpec((1, S, C2), lambda b, n: (b, 0, 0)),
        pl.BlockSpec((1, n_tile, C1), lambda b, n: (b, n, 0)),
        pl.BlockSpec(w1a.shape, lambda b, n: (0, 0)),
        pl.BlockSpec(w1b.shape, lambda b, n: (0, 0)),
        pl.BlockSpec(b1.shape, lambda b, n: (0, 0)),
    ]
    operands = [xyz1, xyz2t, points2, points1, w1a, w1b, b1]
    if has_extra:
        in_specs.append(pl.BlockSpec((1, 1, c0), lambda b, n: (b, 0, 0)))
        operands.append(extra)
    for a in wb:
        in_specs.append(pl.BlockSpec(a.shape, lambda b, n: (0,) * a.ndim))
    operands += wb
    return pl.pallas_call(
        kern,
        grid=(B, N // n_tile),
        in_specs=in_specs,
        out_specs=pl.BlockSpec((1, n_tile, c_last), lambda b, n: (b, n, 0)),
        out_shape=jax.ShapeDtypeStruct((B, N, c_last), jnp.float32),
    )(*operands)


def _fp_broadcast_mlp(points2, points1, layers, n_tile):
    B, N, C1 = points1.shape
    C2 = points2.shape[2]
    nlayers = len(layers)
    c_last = layers[-1]["W"].shape[0]
    w0, b1 = _fold_layer(layers[0])
    w1a, w1b = w0[:C1], w0[C1:]
    wb = []
    for p in layers[1:]:
        wt, b = _fold_layer(p)
        wb += [wt, b]

    def kern(p2_ref, p1_ref, w1a_ref, w1b_ref, b1_ref, *refs):
        out_ref = refs[-1]
        wrefs = refs[:-1]
        t = jnp.dot(p2_ref[0], w1b_ref[...], preferred_element_type=jnp.float32)
        h = (
            jnp.dot(p1_ref[0], w1a_ref[...], preferred_element_type=jnp.float32)
            + t
            + b1_ref[...]
        )
        h = jnp.maximum(h, 0.0)
        out_ref[0] = _mlp_rows(h, wrefs, nlayers - 1)

    in_specs = [
        pl.BlockSpec((1, 1, C2), lambda b, n: (b, 0, 0)),
        pl.BlockSpec((1, n_tile, C1), lambda b, n: (b, n, 0)),
        pl.BlockSpec(w1a.shape, lambda b, n: (0, 0)),
        pl.BlockSpec(w1b.shape, lambda b, n: (0, 0)),
        pl.BlockSpec(b1.shape, lambda b, n: (0, 0)),
    ]
    for a in wb:
        in_specs.append(pl.BlockSpec(a.shape, lambda b, n: (0,) * a.ndim))
    return pl.pallas_call(
        kern,
        grid=(B, N // n_tile),
        in_specs=in_specs,
        out_specs=pl.BlockSpec((1, n_tile, c_last), lambda b, n: (b, n, 0)),
        out_shape=jax.ShapeDtypeStruct((B, N, c_last), jnp.float32),
    )(points2, points1, w1a, w1b, b1, *wb)


def _head(x, conv1, conv2, n_tile):
    B, N, C = x.shape
    w1, b1 = _fold_layer(conv1)
    w2 = conv2["W"].T
    b2 = conv2["b"].reshape(1, -1)
    c_out = w2.shape[1]

    def kern(x_ref, w1_ref, b1_ref, w2_ref, b2_ref, out_ref):
        h = x_ref[0]
        h = jnp.maximum(
            jnp.dot(h, w1_ref[...], preferred_element_type=jnp.float32) + b1_ref[...],
            0.0,
        )
        logits = jnp.dot(h, w2_ref[...], preferred_element_type=jnp.float32) + b2_ref[...]
        m = jnp.max(logits, axis=-1, keepdims=True)
        sh = logits - m
        lse = jnp.log(jnp.sum(jnp.exp(sh), axis=-1, keepdims=True))
        out_ref[0] = sh - lse

    in_specs = [pl.BlockSpec((1, n_tile, C), lambda b, n: (b, n, 0))]
    for a in (w1, b1, w2, b2):
        in_specs.append(pl.BlockSpec(a.shape, lambda b, n: (0,) * a.ndim))
    return pl.pallas_call(
        kern,
        grid=(B, N // n_tile),
        in_specs=in_specs,
        out_specs=pl.BlockSpec((1, n_tile, c_out), lambda b, n: (b, n, 0)),
        out_shape=jax.ShapeDtypeStruct((B, N, c_out), jnp.float32),
    )(x, w1, b1, w2, b2)


def kernel(xyz, cls_label, params):
    B, _, N = xyz.shape
    xyz_t = jnp.transpose(xyz, (0, 2, 1))

    new_xyz1 = _fps(xyz_t, 512)
    pts1 = jnp.concatenate([xyz_t, xyz_t], axis=-1)
    sub1 = jnp.concatenate([new_xyz1, jnp.zeros_like(new_xyz1)], axis=-1)
    l1_points = _sa_group_mlp_max(
        pts1, xyz, new_xyz1, sub1, params["sa1"], 0.2, 32, s_tile=32
    )

    new_xyz2 = _fps(new_xyz1, 128)
    pts2 = jnp.concatenate([new_xyz1, l1_points], axis=-1)
    sub2 = jnp.concatenate([new_xyz2, jnp.zeros((B, 128, 128), jnp.float32)], axis=-1)
    l2_points = _sa_group_mlp_max(
        pts2,
        jnp.transpose(new_xyz1, (0, 2, 1)),
        new_xyz2,
        sub2,
        params["sa2"],
        0.4,
        64,
        s_tile=32,
    )

    g3 = jnp.concatenate([new_xyz2, l2_points], axis=-1)
    l3_points = _sa_all_mlp_max(g3, params["sa3"])

    l2_points = _fp_broadcast_mlp(l3_points, l2_points, params["fp3"], n_tile=128)

    l1_points = _fp_interp_mlp(
        new_xyz1,
        jnp.transpose(new_xyz2, (0, 2, 1)),
        l2_points,
        l1_points,
        params["fp2"],
        n_tile=512,
    )

    w0, b0 = _fold_layer(params["fp1"][0])
    t_cls = jnp.einsum("bc,co->bo", cls_label, w0[:16])[:, None, :]
    prefolded = (w0[16:19] + w0[19:22], w0[22:], b0)
    l0 = _fp_interp_mlp(
        xyz_t,
        jnp.transpose(new_xyz1, (0, 2, 1)),
        l1_points,
        xyz_t,
        [prefolded] + params["fp1"][1:],
        n_tile=512,
        extra=t_cls,
    )

    x = _head(l0, params["conv1"], params["conv2"], n_tile=512)
    return x, jnp.transpose(l3_points, (0, 2, 1))
